# Initial kernel scaffold; baseline (speedup 1.0000x reference)
import jax
import jax.numpy as jnp
from jax.experimental import pallas as pl


def kernel(x, edge_index, edge_attr, W1a, b1a, W1b, b1b, root1, bias1, W2a, b2a, W2b, b2b, root2, bias2, Wl, bl):
    raise NotImplementedError("write your pallas kernel here")


# speedup vs baseline: 3.2937x; 3.2937x over previous
import functools

import jax
import jax.numpy as jnp
from jax import lax
from jax.experimental import pallas as pl
from jax.experimental.pallas import tpu as pltpu
from jax.experimental.pallas import tpu_sc as plsc

N = 10000
E = 320000
IN = 16
H = 16
EA = 4

NC = 2
NS = 16
NW = NC * NS
PERW = E // NW
CH = 80
NCHUNK = PERW // CH
ZROWS = N // NS

@functools.cache
def _mesh():
  return plsc.VectorSubcoreMesh(core_axis_name="c", subcore_axis_name="s")


_SC_PARAMS = pltpu.CompilerParams(use_tc_tiling_on_sc=False)


def _sc_gather(table, idx3):
  @functools.partial(
      pl.kernel,
      mesh=_mesh(),
      out_type=jax.ShapeDtypeStruct((E, 16), jnp.float32),
      scratch_types=[
          pltpu.VMEM((NCHUNK, CH), jnp.int32),
          pltpu.VMEM((CH, 16), jnp.float32),
          pltpu.SemaphoreType.DMA,
      ],
      compiler_params=_SC_PARAMS,
  )
  def k(table_hbm, idx_hbm, out_hbm, idx_v, rows_v, sem):
    wid = lax.axis_index("s") * NC + lax.axis_index("c")
    base = wid * PERW
    pltpu.sync_copy(idx_hbm.at[wid], idx_v)

    def body(j, carry):
      pltpu.async_copy(table_hbm.at[idx_v.at[j]], rows_v, sem).wait()
      pltpu.sync_copy(rows_v, out_hbm.at[pl.ds(base + j * CH, CH)])
      return carry

    lax.fori_loop(0, NCHUNK, body, 0)

  return k(table, idx3)


def _sc_scatter(msg, idx3, with_cnt):
  outs = [jax.ShapeDtypeStruct((NC, N, 16), jnp.float32)]
  scratch = [
      pltpu.VMEM((NCHUNK, CH), jnp.int32),
      pltpu.VMEM((CH, 16), jnp.float32),
      pltpu.VMEM((ZROWS, 16), jnp.float32),
      pltpu.VMEM_SHARED((N, 16), jnp.float32),
      pltpu.SemaphoreType.DMA,
  ]
  if with_cnt:
    outs.append(jax.ShapeDtypeStruct((NC, N, 16), jnp.float32))
    scratch.insert(2, pltpu.VMEM((CH, 16), jnp.float32))
    scratch.insert(4, pltpu.VMEM_SHARED((N, 16), jnp.float32))

  @functools.partial(
      pl.kernel, mesh=_mesh(), out_type=outs, scratch_types=scratch,
      compiler_params=_SC_PARAMS)
  def k(*refs):
    if with_cnt:
      (msg_hbm, idx_hbm, out_sum, out_cnt,
       idx_v, rows_v, ones_v, stage_v, acc, cacc, sem) = refs
    else:
      (msg_hbm, idx_hbm, out_sum,
       idx_v, rows_v, stage_v, acc, sem) = refs
    sid = lax.axis_index("s")
    cid = lax.axis_index("c")
    wid = sid * NC + cid
    base = wid * PERW

    def zbody(r, carry):
      stage_v[r, :] = jnp.zeros((16,), jnp.float32)
      return carry

    lax.fori_loop(0, ZROWS, zbody, 0)
    pltpu.sync_copy(stage_v, acc.at[pl.ds(sid * ZROWS, ZROWS)])
    if with_cnt:
      pltpu.sync_copy(stage_v, cacc.at[pl.ds(sid * ZROWS, ZROWS)])

      def obody(r, carry):
        ones_v[r, :] = jnp.ones((16,), jnp.float32)
        return carry

      lax.fori_loop(0, CH, obody, 0)
    pltpu.sync_copy(idx_hbm.at[wid], idx_v)
    plsc.subcore_barrier()

    def body(j, carry):
      pltpu.sync_copy(msg_hbm.at[pl.ds(base + j * CH, CH)], rows_v)
      pltpu.sync_copy(rows_v, acc.at[idx_v.at[j]], add=True)
      if with_cnt:
        pltpu.sync_copy(ones_v, cacc.at[idx_v.at[j]], add=True)
      return carry

    lax.fori_loop(0, NCHUNK, body, 0)
    plsc.subcore_barrier()

    pltpu.sync_copy(acc.at[pl.ds(sid * ZROWS, ZROWS)], stage_v)
    pltpu.sync_copy(stage_v, out_sum.at[cid, pl.ds(sid * ZROWS, ZROWS)])
    if with_cnt:
      pltpu.sync_copy(cacc.at[pl.ds(sid * ZROWS, ZROWS)], stage_v)
      pltpu.sync_copy(stage_v, out_cnt.at[cid, pl.ds(sid * ZROWS, ZROWS)])

  res = k(msg, idx3)
  return res if with_cnt else res[0]


_TB = 3200


def _tc_edge_msgs(ea, xj, Wa, ba, Wb, bb, R, S):
  def body(ea_ref, xj_ref, wa, ba_r, wb, bb_r, r_r, s_r, out_ref):
    h = jnp.maximum(
        jnp.dot(ea_ref[...], wa[...], preferred_element_type=jnp.float32)
        + ba_r[...], 0.0)
    w = jnp.dot(h, wb[...], preferred_element_type=jnp.float32) + bb_r[...]
    xe = jnp.dot(xj_ref[...], r_r[...], preferred_element_type=jnp.float32)
    out_ref[...] = jnp.dot(
        w * xe, s_r[...], preferred_element_type=jnp.float32)

  zero = lambda i: (0, 0)
  return pl.pallas_call(
      body,
      grid=(E // _TB,),
      in_specs=[
          pl.BlockSpec((_TB, EA), lambda i: (i, 0)),
          pl.BlockSpec((_TB, IN), lambda i: (i, 0)),
          pl.BlockSpec((EA, 32), zero),
          pl.BlockSpec((1, 32), zero),
          pl.BlockSpec((32, IN * H), zero),
          pl.BlockSpec((1, IN * H), zero),
          pl.BlockSpec((IN, IN * H), zero),
          pl.BlockSpec((IN * H, H), zero),
      ],
      out_specs=pl.BlockSpec((_TB, H), lambda i: (i, 0)),
      out_shape=jax.ShapeDtypeStruct((E, H), jnp.float32),
  )(ea, xj, Wa, ba.reshape(1, 32), Wb, bb.reshape(1, IN * H), R, S)


def _tc_finalize1(sums, cnts, x, root, bias):
  def body(s_ref, c_ref, x_ref, r_ref, b_ref, h_ref, rinv_ref):
    cnt = c_ref[0] + c_ref[1]
    rinv = 1.0 / jnp.maximum(cnt, 1.0)
    mean = (s_ref[0] + s_ref[1]) * rinv
    h = mean + jnp.dot(
        x_ref[...], r_ref[...], preferred_element_type=jnp.float32) + b_ref[...]
    h_ref[...] = jnp.maximum(h, 0.0)
    rinv_ref[...] = rinv

  return pl.pallas_call(
      body,
      out_shape=[
          jax.ShapeDtypeStruct((N, H), jnp.float32),
          jax.ShapeDtypeStruct((N, H), jnp.float32),
      ],
  )(sums, cnts, x, root, bias.reshape(1, H))


def _tc_finalize2(sums, rinv, h1, root, bias, Wl, bl):
  def body(s_ref, rinv_ref, h1_ref, r_ref, b_ref, wl_ref, bl_ref, out_ref):
    mean = (s_ref[0] + s_ref[1]) * rinv_ref[...]
    h2 = mean + jnp.dot(
        h1_ref[...], r_ref[...], preferred_element_type=jnp.float32) + b_ref[...]
    h2 = jnp.maximum(h2, 0.0)
    out_ref[...] = jnp.dot(
        h2, wl_ref[...], preferred_element_type=jnp.float32) + bl_ref[...]

  return pl.pallas_call(
      body,
      out_shape=jax.ShapeDtypeStruct((N, 1), jnp.float32),
  )(sums, rinv, h1, root, bias.reshape(1, H), Wl, bl.reshape(1, 1))


def kernel(x, edge_index, edge_attr, W1a, b1a, W1b, b1b, root1, bias1,
           W2a, b2a, W2b, b2b, root2, bias2, Wl, bl):
  src3 = edge_index[0].astype(jnp.int32).reshape(NW, NCHUNK, CH)
  dst3 = edge_index[1].astype(jnp.int32).reshape(NW, NCHUNK, CH)

  c = jnp.arange(IN * H, dtype=jnp.int32)
  R = (jnp.arange(IN, dtype=jnp.int32)[:, None] == (c // H)[None, :]
       ).astype(jnp.float32)
  S = ((c % H)[:, None] == jnp.arange(H, dtype=jnp.int32)[None, :]
       ).astype(jnp.float32)

  xj = _sc_gather(x, src3)
  msg1 = _tc_edge_msgs(edge_attr, xj, W1a, b1a, W1b, b1b, R, S)
  sums1, cnts = _sc_scatter(msg1, dst3, True)
  h1, rinv = _tc_finalize1(sums1, cnts, x, root1, bias1)

  xj2 = _sc_gather(h1, src3)
  msg2 = _tc_edge_msgs(edge_attr, xj2, W2a, b2a, W2b, b2b, R, S)
  sums2 = _sc_scatter(msg2, dst3, False)
  out = _tc_finalize2(sums2, rinv, h1, root2, bias2, Wl, bl)
  return out[:, 0]


# speedup vs baseline: 3.9640x; 1.2035x over previous
import functools

import jax
import jax.numpy as jnp
from jax import lax
from jax.experimental import pallas as pl
from jax.experimental.pallas import tpu as pltpu
from jax.experimental.pallas import tpu_sc as plsc

N = 10000
E = 320000
IN = 16
H = 16
EA = 4

NC = 2
NS = 16
NW = NC * NS
PERW = E // NW
CH = 80
NCHUNK = PERW // CH
GRP = 5
NGRP = NCHUNK // GRP
ZROWS = N // NS

@functools.cache
def _mesh():
  return plsc.VectorSubcoreMesh(core_axis_name="c", subcore_axis_name="s")


_SC_PARAMS = pltpu.CompilerParams(use_tc_tiling_on_sc=False)


def _sc_gather(table, idx3):
  @functools.partial(
      pl.kernel,
      mesh=_mesh(),
      out_type=jax.ShapeDtypeStruct((E, 16), jnp.float32),
      scratch_types=[
          pltpu.VMEM((NCHUNK, CH), jnp.int32),
          pltpu.VMEM((GRP, CH, 16), jnp.float32),
          pltpu.SemaphoreType.DMA,
          pltpu.SemaphoreType.DMA,
      ],
      compiler_params=_SC_PARAMS,
  )
  def k(table_hbm, idx_hbm, out_hbm, idx_v, rows_v, gsem, ssem):
    wid = lax.axis_index("s") * NC + lax.axis_index("c")
    base = wid * PERW
    pltpu.sync_copy(idx_hbm.at[wid], idx_v)

    def group(g, carry):
      j0 = g * GRP

      @pl.when(g > 0)
      def _drain_stores():
        for b in range(GRP):
          pltpu.make_async_copy(
              rows_v.at[b], out_hbm.at[pl.ds(base, CH)], ssem).wait()

      for b in range(GRP):
        pltpu.async_copy(table_hbm.at[idx_v.at[j0 + b]], rows_v.at[b], gsem)
      for b in range(GRP):
        pltpu.make_async_copy(
            table_hbm.at[idx_v.at[j0 + b]], rows_v.at[b], gsem).wait()
      for b in range(GRP):
        pltpu.async_copy(
            rows_v.at[b], out_hbm.at[pl.ds(base + (j0 + b) * CH, CH)], ssem)
      return carry

    lax.fori_loop(0, NGRP, group, 0)
    for b in range(GRP):
      pltpu.make_async_copy(
          rows_v.at[b], out_hbm.at[pl.ds(base, CH)], ssem).wait()

  return k(table, idx3)


def _sc_scatter(msg, idx3, with_cnt):
  outs = [jax.ShapeDtypeStruct((NC, N, 16), jnp.float32)]
  scratch = [
      pltpu.VMEM((NCHUNK, CH), jnp.int32),
      pltpu.VMEM((GRP, CH, 16), jnp.float32),
      pltpu.VMEM((ZROWS, 16), jnp.float32),
      pltpu.VMEM_SHARED((N, 16), jnp.float32),
      pltpu.SemaphoreType.DMA,
      pltpu.SemaphoreType.DMA,
      pltpu.SemaphoreType.DMA,
  ]
  if with_cnt:
    outs.append(jax.ShapeDtypeStruct((NC, N, 16), jnp.float32))
    scratch.insert(2, pltpu.VMEM((CH, 16), jnp.float32))
    scratch.insert(4, pltpu.VMEM_SHARED((N, 16), jnp.float32))

  @functools.partial(
      pl.kernel, mesh=_mesh(), out_type=outs, scratch_types=scratch,
      compiler_params=_SC_PARAMS)
  def k(*refs):
    if with_cnt:
      (msg_hbm, idx_hbm, out_sum, out_cnt,
       idx_v, rows_v, ones_v, stage_v, acc, cacc, lsem, asem, csem) = refs
    else:
      (msg_hbm, idx_hbm, out_sum,
       idx_v, rows_v, stage_v, acc, lsem, asem, csem) = refs
    sid = lax.axis_index("s")
    cid = lax.axis_index("c")
    wid = sid * NC + cid
    base = wid * PERW

    def zbody(r, carry):
      stage_v[r, :] = jnp.zeros((16,), jnp.float32)
      return carry

    lax.fori_loop(0, ZROWS, zbody, 0)
    pltpu.sync_copy(stage_v, acc.at[pl.ds(sid * ZROWS, ZROWS)])
    if with_cnt:
      pltpu.sync_copy(stage_v, cacc.at[pl.ds(sid * ZROWS, ZROWS)])

      def obody(r, carry):
        ones_v[r, :] = jnp.ones((16,), jnp.float32)
        return carry

      lax.fori_loop(0, CH, obody, 0)
    pltpu.sync_copy(idx_hbm.at[wid], idx_v)
    plsc.subcore_barrier()

    def group(g, carry):
      j0 = g * GRP

      @pl.when(g > 0)
      def _drain_adds():
        for b in range(GRP):
          pltpu.make_async_copy(
              rows_v.at[b], acc.at[idx_v.at[j0 + b]], asem).wait()
          if with_cnt:
            pltpu.make_async_copy(
                ones_v, cacc.at[idx_v.at[j0 + b]], csem).wait()

      for b in range(GRP):
        pltpu.async_copy(
            msg_hbm.at[pl.ds(base + (j0 + b) * CH, CH)], rows_v.at[b], lsem)
      for b in range(GRP):
        pltpu.make_async_copy(
            msg_hbm.at[pl.ds(base, CH)], rows_v.at[b], lsem).wait()
      for b in range(GRP):
        pltpu.async_copy(rows_v.at[b], acc.at[idx_v.at[j0 + b]], asem,
                         add=True)
        if with_cnt:
          pltpu.async_copy(ones_v, cacc.at[idx_v.at[j0 + b]], csem, add=True)
      return carry

    lax.fori_loop(0, NGRP, group, 0)
    for b in range(GRP):
      pltpu.make_async_copy(rows_v.at[b], acc.at[idx_v.at[b]], asem).wait()
      if with_cnt:
        pltpu.make_async_copy(ones_v, cacc.at[idx_v.at[b]], csem).wait()
    plsc.subcore_barrier()

    pltpu.sync_copy(acc.at[pl.ds(sid * ZROWS, ZROWS)], stage_v)
    pltpu.sync_copy(stage_v, out_sum.at[cid, pl.ds(sid * ZROWS, ZROWS)])
    if with_cnt:
      pltpu.sync_copy(cacc.at[pl.ds(sid * ZROWS, ZROWS)], stage_v)
      pltpu.sync_copy(stage_v, out_cnt.at[cid, pl.ds(sid * ZROWS, ZROWS)])

  res = k(msg, idx3)
  return res if with_cnt else res[0]


_TB = 3200


def _tc_edge_msgs(ea, xj, Wa, ba, Wb, bb, R, S):
  def body(ea_ref, xj_ref, wa, ba_r, wb, bb_r, r_r, s_r, out_ref):
    h = jnp.maximum(
        jnp.dot(ea_ref[...], wa[...], preferred_element_type=jnp.float32)
        + ba_r[...], 0.0)
    w = jnp.dot(h, wb[...], preferred_element_type=jnp.float32) + bb_r[...]
    xe = jnp.dot(xj_ref[...], r_r[...], preferred_element_type=jnp.float32)
    out_ref[...] = jnp.dot(
        w * xe, s_r[...], preferred_element_type=jnp.float32)

  zero = lambda i: (0, 0)
  return pl.pallas_call(
      body,
      grid=(E // _TB,),
      in_specs=[
          pl.BlockSpec((_TB, EA), lambda i: (i, 0)),
          pl.BlockSpec((_TB, IN), lambda i: (i, 0)),
          pl.BlockSpec((EA, 32), zero),
          pl.BlockSpec((1, 32), zero),
          pl.BlockSpec((32, IN * H), zero),
          pl.BlockSpec((1, IN * H), zero),
          pl.BlockSpec((IN, IN * H), zero),
          pl.BlockSpec((IN * H, H), zero),
      ],
      out_specs=pl.BlockSpec((_TB, H), lambda i: (i, 0)),
      out_shape=jax.ShapeDtypeStruct((E, H), jnp.float32),
  )(ea, xj, Wa, ba.reshape(1, 32), Wb, bb.reshape(1, IN * H), R, S)


def _tc_finalize1(sums, cnts, x, root, bias):
  def body(s_ref, c_ref, x_ref, r_ref, b_ref, h_ref, rinv_ref):
    cnt = c_ref[0] + c_ref[1]
    rinv = 1.0 / jnp.maximum(cnt, 1.0)
    mean = (s_ref[0] + s_ref[1]) * rinv
    h = mean + jnp.dot(
        x_ref[...], r_ref[...], preferred_element_type=jnp.float32) + b_ref[...]
    h_ref[...] = jnp.maximum(h, 0.0)
    rinv_ref[...] = rinv

  return pl.pallas_call(
      body,
      out_shape=[
          jax.ShapeDtypeStruct((N, H), jnp.float32),
          jax.ShapeDtypeStruct((N, H), jnp.float32),
      ],
  )(sums, cnts, x, root, bias.reshape(1, H))


def _tc_finalize2(sums, rinv, h1, root, bias, Wl, bl):
  def body(s_ref, rinv_ref, h1_ref, r_ref, b_ref, wl_ref, bl_ref, out_ref):
    mean = (s_ref[0] + s_ref[1]) * rinv_ref[...]
    h2 = mean + jnp.dot(
        h1_ref[...], r_ref[...], preferred_element_type=jnp.float32) + b_ref[...]
    h2 = jnp.maximum(h2, 0.0)
    out_ref[...] = jnp.dot(
        h2, wl_ref[...], preferred_element_type=jnp.float32) + bl_ref[...]

  return pl.pallas_call(
      body,
      out_shape=jax.ShapeDtypeStruct((N, 1), jnp.float32),
  )(sums, rinv, h1, root, bias.reshape(1, H), Wl, bl.reshape(1, 1))


def kernel(x, edge_index, edge_attr, W1a, b1a, W1b, b1b, root1, bias1,
           W2a, b2a, W2b, b2b, root2, bias2, Wl, bl):
  src3 = edge_index[0].astype(jnp.int32).reshape(NW, NCHUNK, CH)
  dst3 = edge_index[1].astype(jnp.int32).reshape(NW, NCHUNK, CH)

  c = jnp.arange(IN * H, dtype=jnp.int32)
  R = (jnp.arange(IN, dtype=jnp.int32)[:, None] == (c // H)[None, :]
       ).astype(jnp.float32)
  S = ((c % H)[:, None] == jnp.arange(H, dtype=jnp.int32)[None, :]
       ).astype(jnp.float32)

  xj = _sc_gather(x, src3)
  msg1 = _tc_edge_msgs(edge_attr, xj, W1a, b1a, W1b, b1b, R, S)
  sums1, cnts = _sc_scatter(msg1, dst3, True)
  h1, rinv = _tc_finalize1(sums1, cnts, x, root1, bias1)

  xj2 = _sc_gather(h1, src3)
  msg2 = _tc_edge_msgs(edge_attr, xj2, W2a, b2a, W2b, b2b, R, S)
  sums2 = _sc_scatter(msg2, dst3, False)
  out = _tc_finalize2(sums2, rinv, h1, root2, bias2, Wl, bl)
  return out[:, 0]


# speedup vs baseline: 5.8709x; 1.4811x over previous
import functools

import jax
import jax.numpy as jnp
from jax import lax
from jax.experimental import pallas as pl
from jax.experimental.pallas import tpu as pltpu
from jax.experimental.pallas import tpu_sc as plsc

N = 10000
E = 320000
IN = 16
H = 16
EA = 4

NC = 2
NS = 16
NW = NC * NS
PERW = E // NW
CH = 80
NCHUNK = PERW // CH
GRP = 5
NGRP = NCHUNK // GRP
ZROWS = N // NS

@functools.cache
def _mesh():
  return plsc.VectorSubcoreMesh(core_axis_name="c", subcore_axis_name="s")


_SC_PARAMS = pltpu.CompilerParams(use_tc_tiling_on_sc=False)


def _sc_gather(table, idx3):
  @functools.partial(
      pl.kernel,
      mesh=_mesh(),
      out_type=jax.ShapeDtypeStruct((E, 16), jnp.float32),
      scratch_types=[
          pltpu.VMEM((NCHUNK, CH), jnp.int32),
          pltpu.VMEM((GRP, CH, 16), jnp.float32),
          pltpu.SemaphoreType.DMA,
          pltpu.SemaphoreType.DMA,
      ],
      compiler_params=_SC_PARAMS,
  )
  def k(table_hbm, idx_hbm, out_hbm, idx_v, rows_v, gsem, ssem):
    wid = lax.axis_index("s") * NC + lax.axis_index("c")
    base = wid * PERW
    pltpu.sync_copy(idx_hbm.at[wid], idx_v)

    def group(g, carry):
      j0 = g * GRP

      @pl.when(g > 0)
      def _drain_stores():
        for b in range(GRP):
          pltpu.make_async_copy(
              rows_v.at[b], out_hbm.at[pl.ds(base, CH)], ssem).wait()

      for b in range(GRP):
        pltpu.async_copy(table_hbm.at[idx_v.at[j0 + b]], rows_v.at[b], gsem)
      for b in range(GRP):
        pltpu.make_async_copy(
            table_hbm.at[idx_v.at[j0 + b]], rows_v.at[b], gsem).wait()
      for b in range(GRP):
        pltpu.async_copy(
            rows_v.at[b], out_hbm.at[pl.ds(base + (j0 + b) * CH, CH)], ssem)
      return carry

    lax.fori_loop(0, NGRP, group, 0)
    for b in range(GRP):
      pltpu.make_async_copy(
          rows_v.at[b], out_hbm.at[pl.ds(base, CH)], ssem).wait()

  return k(table, idx3)


def _sc_scatter(msg, idx3, with_cnt):
  outs = [jax.ShapeDtypeStruct((NC, N, 16), jnp.float32)]
  scratch = [
      pltpu.VMEM((NCHUNK, CH), jnp.int32),
      pltpu.VMEM((GRP, CH, 16), jnp.float32),
      pltpu.VMEM((ZROWS, 16), jnp.float32),
      pltpu.VMEM_SHARED((N, 16), jnp.float32),
      pltpu.SemaphoreType.DMA,
      pltpu.SemaphoreType.DMA,
      pltpu.SemaphoreType.DMA,
  ]
  if with_cnt:
    outs.append(jax.ShapeDtypeStruct((NC, N, 16), jnp.float32))
    scratch.insert(2, pltpu.VMEM((CH, 16), jnp.float32))
    scratch.insert(4, pltpu.VMEM_SHARED((N, 16), jnp.float32))

  @functools.partial(
      pl.kernel, mesh=_mesh(), out_type=outs, scratch_types=scratch,
      compiler_params=_SC_PARAMS)
  def k(*refs):
    if with_cnt:
      (msg_hbm, idx_hbm, out_sum, out_cnt,
       idx_v, rows_v, ones_v, stage_v, acc, cacc, lsem, asem, csem) = refs
    else:
      (msg_hbm, idx_hbm, out_sum,
       idx_v, rows_v, stage_v, acc, lsem, asem, csem) = refs
    sid = lax.axis_index("s")
    cid = lax.axis_index("c")
    wid = sid * NC + cid
    base = wid * PERW

    def zbody(r, carry):
      stage_v[r, :] = jnp.zeros((16,), jnp.float32)
      return carry

    lax.fori_loop(0, ZROWS, zbody, 0)
    pltpu.sync_copy(stage_v, acc.at[pl.ds(sid * ZROWS, ZROWS)])
    if with_cnt:
      pltpu.sync_copy(stage_v, cacc.at[pl.ds(sid * ZROWS, ZROWS)])

      def obody(r, carry):
        ones_v[r, :] = jnp.ones((16,), jnp.float32)
        return carry

      lax.fori_loop(0, CH, obody, 0)
    pltpu.sync_copy(idx_hbm.at[wid], idx_v)
    plsc.subcore_barrier()

    def group(g, carry):
      j0 = g * GRP

      @pl.when(g > 0)
      def _drain_adds():
        for b in range(GRP):
          pltpu.make_async_copy(
              rows_v.at[b], acc.at[idx_v.at[j0 + b]], asem).wait()
          if with_cnt:
            pltpu.make_async_copy(
                ones_v, cacc.at[idx_v.at[j0 + b]], csem).wait()

      for b in range(GRP):
        pltpu.async_copy(
            msg_hbm.at[pl.ds(base + (j0 + b) * CH, CH)], rows_v.at[b], lsem)
      for b in range(GRP):
        pltpu.make_async_copy(
            msg_hbm.at[pl.ds(base, CH)], rows_v.at[b], lsem).wait()
      for b in range(GRP):
        pltpu.async_copy(rows_v.at[b], acc.at[idx_v.at[j0 + b]], asem,
                         add=True)
        if with_cnt:
          pltpu.async_copy(ones_v, cacc.at[idx_v.at[j0 + b]], csem, add=True)
      return carry

    lax.fori_loop(0, NGRP, group, 0)
    for b in range(GRP):
      pltpu.make_async_copy(rows_v.at[b], acc.at[idx_v.at[b]], asem).wait()
      if with_cnt:
        pltpu.make_async_copy(ones_v, cacc.at[idx_v.at[b]], csem).wait()
    plsc.subcore_barrier()

    pltpu.sync_copy(acc.at[pl.ds(sid * ZROWS, ZROWS)], stage_v)
    pltpu.sync_copy(stage_v, out_sum.at[cid, pl.ds(sid * ZROWS, ZROWS)])
    if with_cnt:
      pltpu.sync_copy(cacc.at[pl.ds(sid * ZROWS, ZROWS)], stage_v)
      pltpu.sync_copy(stage_v, out_cnt.at[cid, pl.ds(sid * ZROWS, ZROWS)])

  res = k(msg, idx3)
  return res if with_cnt else res[0]


_TB = 3200


def _tc_edge_msgs(eap, xjp, Wa_bd, ba8, Wb_bd, bb8, R_bd, S_bd):
  TB8 = _TB // 8

  def body(eap_ref, xj_ref, wa, ba_r, wb, bb_r, r_r, s_r, out_ref):
    hp = jnp.maximum(
        jnp.dot(eap_ref[...], wa[...], preferred_element_type=jnp.float32)
        + ba_r[...], 0.0)
    wp = jnp.dot(hp, wb[...], preferred_element_type=jnp.float32) + bb_r[...]
    xep = jnp.dot(xj_ref[...], r_r[...], preferred_element_type=jnp.float32)
    out_ref[...] = jnp.dot(
        wp * xep, s_r[...], preferred_element_type=jnp.float32)

  zero = lambda i: (0, 0)
  return pl.pallas_call(
      body,
      grid=(E // _TB,),
      in_specs=[
          pl.BlockSpec((TB8, 8 * EA), lambda i: (i, 0)),
          pl.BlockSpec((TB8, 128), lambda i: (i, 0)),
          pl.BlockSpec((8 * EA, 8 * 32), zero),
          pl.BlockSpec((1, 8 * 32), zero),
          pl.BlockSpec((8 * 32, 8 * IN * H), zero),
          pl.BlockSpec((1, 8 * IN * H), zero),
          pl.BlockSpec((128, 8 * IN * H), zero),
          pl.BlockSpec((8 * IN * H, 128), zero),
      ],
      out_specs=pl.BlockSpec((TB8, 128), lambda i: (i, 0)),
      out_shape=jax.ShapeDtypeStruct((E // 8, 128), jnp.float32),
  )(eap, xjp, Wa_bd, ba8.reshape(1, 8 * 32), Wb_bd,
    bb8.reshape(1, 8 * IN * H), R_bd, S_bd)


NP8 = N // 8


def _tc_finalize1(sums_p, cnts_p, x_p, root_bd, bias_t):
  def body(s_ref, c_ref, x_ref, r_ref, b_ref, h_ref, rinv_ref):
    cnt = c_ref[0] + c_ref[1]
    rinv = 1.0 / jnp.maximum(cnt, 1.0)
    mean = (s_ref[0] + s_ref[1]) * rinv
    h = mean + jnp.dot(
        x_ref[...], r_ref[...], preferred_element_type=jnp.float32) + b_ref[...]
    h_ref[...] = jnp.maximum(h, 0.0)
    rinv_ref[...] = rinv

  return pl.pallas_call(
      body,
      out_shape=[
          jax.ShapeDtypeStruct((NP8, 128), jnp.float32),
          jax.ShapeDtypeStruct((NP8, 128), jnp.float32),
      ],
  )(sums_p, cnts_p, x_p, root_bd, bias_t.reshape(1, 128))


def _tc_finalize2(sums_p, rinv_p, h1_p, root_bd, bias_t, Wl_bd, bl_t):
  def body(s_ref, rinv_ref, h1_ref, r_ref, b_ref, wl_ref, bl_ref, out_ref):
    mean = (s_ref[0] + s_ref[1]) * rinv_ref[...]
    h2 = mean + jnp.dot(
        h1_ref[...], r_ref[...], preferred_element_type=jnp.float32) + b_ref[...]
    h2 = jnp.maximum(h2, 0.0)
    out_ref[...] = jnp.dot(
        h2, wl_ref[...], preferred_element_type=jnp.float32) + bl_ref[...]

  return pl.pallas_call(
      body,
      out_shape=jax.ShapeDtypeStruct((NP8, 8), jnp.float32),
  )(sums_p, rinv_p, h1_p, root_bd, bias_t.reshape(1, 128), Wl_bd,
    bl_t.reshape(1, 8))


def kernel(x, edge_index, edge_attr, W1a, b1a, W1b, b1b, root1, bias1,
           W2a, b2a, W2b, b2b, root2, bias2, Wl, bl):
  src3 = edge_index[0].astype(jnp.int32).reshape(NW, NCHUNK, CH)
  dst3 = edge_index[1].astype(jnp.int32).reshape(NW, NCHUNK, CH)

  c = jnp.arange(IN * H, dtype=jnp.int32)
  R = (jnp.arange(IN, dtype=jnp.int32)[:, None] == (c // H)[None, :]
       ).astype(jnp.float32)
  S = ((c % H)[:, None] == jnp.arange(H, dtype=jnp.int32)[None, :]
       ).astype(jnp.float32)

  eye8 = jnp.eye(8, dtype=jnp.float32)
  W1a_bd = jnp.kron(eye8, W1a)
  W1b_bd = jnp.kron(eye8, W1b)
  W2a_bd = jnp.kron(eye8, W2a)
  W2b_bd = jnp.kron(eye8, W2b)
  R_bd = jnp.kron(eye8, R)
  S_bd = jnp.kron(eye8, S)
  b1a8 = jnp.tile(b1a, 8)
  b1b8 = jnp.tile(b1b, 8)
  b2a8 = jnp.tile(b2a, 8)
  b2b8 = jnp.tile(b2b, 8)
  root1_bd = jnp.kron(eye8, root1)
  root2_bd = jnp.kron(eye8, root2)
  Wl_bd = jnp.kron(eye8, Wl)
  bias1_t = jnp.tile(bias1, 8)
  bias2_t = jnp.tile(bias2, 8)
  bl_t = jnp.tile(bl, 8)
  eap = edge_attr.reshape(E // 8, 8 * EA)

  xj = _sc_gather(x, src3)
  msg1 = _tc_edge_msgs(eap, xj.reshape(E // 8, 128),
                       W1a_bd, b1a8, W1b_bd, b1b8, R_bd, S_bd)
  sums1, cnts = _sc_scatter(msg1.reshape(E, 16), dst3, True)
  h1p, rinvp = _tc_finalize1(sums1.reshape(NC, NP8, 128),
                             cnts.reshape(NC, NP8, 128),
                             x.reshape(NP8, 128), root1_bd, bias1_t)

  xj2 = _sc_gather(h1p.reshape(N, H), src3)
  msg2 = _tc_edge_msgs(eap, xj2.reshape(E // 8, 128),
                       W2a_bd, b2a8, W2b_bd, b2b8, R_bd, S_bd)
  sums2 = _sc_scatter(msg2.reshape(E, 16), dst3, False)
  out = _tc_finalize2(sums2.reshape(NC, NP8, 128), rinvp, h1p,
                      root2_bd, bias2_t, Wl_bd, bl_t)
  return out.reshape(N)


# speedup vs baseline: 6.2111x; 1.0580x over previous
import functools

import jax
import jax.numpy as jnp
from jax import lax
from jax.experimental import pallas as pl
from jax.experimental.pallas import tpu as pltpu
from jax.experimental.pallas import tpu_sc as plsc

N = 10000
E = 320000
IN = 16
H = 16
EA = 4

NC = 2
NS = 16
NW = NC * NS
PERW = E // NW
CH = 80
NCHUNK = PERW // CH
GRP = 5
NGRP = NCHUNK // GRP
ZROWS = N // NS

@functools.cache
def _mesh():
  return plsc.VectorSubcoreMesh(core_axis_name="c", subcore_axis_name="s")


_SC_PARAMS = pltpu.CompilerParams(use_tc_tiling_on_sc=False)


def _sc_gather(table, idx3):
  @functools.partial(
      pl.kernel,
      mesh=_mesh(),
      out_type=jax.ShapeDtypeStruct((E, 16), jnp.float32),
      scratch_types=[
          pltpu.VMEM((NCHUNK, CH), jnp.int32),
          pltpu.VMEM((GRP, CH, 16), jnp.float32),
          pltpu.SemaphoreType.DMA,
          pltpu.SemaphoreType.DMA,
      ],
      compiler_params=_SC_PARAMS,
  )
  def k(table_hbm, idx_hbm, out_hbm, idx_v, rows_v, gsem, ssem):
    wid = lax.axis_index("s") * NC + lax.axis_index("c")
    base = wid * PERW
    pltpu.sync_copy(idx_hbm.at[wid], idx_v)

    def group(g, carry):
      j0 = g * GRP

      @pl.when(g > 0)
      def _drain_stores():
        for b in range(GRP):
          pltpu.make_async_copy(
              rows_v.at[b], out_hbm.at[pl.ds(base, CH)], ssem).wait()

      for b in range(GRP):
        pltpu.async_copy(table_hbm.at[idx_v.at[j0 + b]], rows_v.at[b], gsem)
      for b in range(GRP):
        pltpu.make_async_copy(
            table_hbm.at[idx_v.at[j0 + b]], rows_v.at[b], gsem).wait()
      for b in range(GRP):
        pltpu.async_copy(
            rows_v.at[b], out_hbm.at[pl.ds(base + (j0 + b) * CH, CH)], ssem)
      return carry

    lax.fori_loop(0, NGRP, group, 0)
    for b in range(GRP):
      pltpu.make_async_copy(
          rows_v.at[b], out_hbm.at[pl.ds(base, CH)], ssem).wait()

  return k(table, idx3)


def _sc_scatter(msg, idx3, with_cnt):
  outs = [jax.ShapeDtypeStruct((NC, N, 16), jnp.float32)]
  scratch = [
      pltpu.VMEM((NCHUNK, CH), jnp.int32),
      pltpu.VMEM((GRP, CH, 16), jnp.float32),
      pltpu.VMEM((ZROWS, 16), jnp.float32),
      pltpu.VMEM_SHARED((N, 16), jnp.float32),
      pltpu.SemaphoreType.DMA,
      pltpu.SemaphoreType.DMA,
      pltpu.SemaphoreType.DMA,
  ]
  if with_cnt:
    outs.append(jax.ShapeDtypeStruct((NC, N, 16), jnp.float32))
    scratch.insert(2, pltpu.VMEM((CH, 16), jnp.float32))
    scratch.insert(4, pltpu.VMEM_SHARED((N, 16), jnp.float32))

  @functools.partial(
      pl.kernel, mesh=_mesh(), out_type=outs, scratch_types=scratch,
      compiler_params=_SC_PARAMS)
  def k(*refs):
    if with_cnt:
      (msg_hbm, idx_hbm, out_sum, out_cnt,
       idx_v, rows_v, ones_v, stage_v, acc, cacc, lsem, asem, csem) = refs
    else:
      (msg_hbm, idx_hbm, out_sum,
       idx_v, rows_v, stage_v, acc, lsem, asem, csem) = refs
    sid = lax.axis_index("s")
    cid = lax.axis_index("c")
    wid = sid * NC + cid
    base = wid * PERW

    def zbody(r, carry):
      stage_v[r, :] = jnp.zeros((16,), jnp.float32)
      return carry

    lax.fori_loop(0, ZROWS, zbody, 0)
    pltpu.sync_copy(stage_v, acc.at[pl.ds(sid * ZROWS, ZROWS)])
    if with_cnt:
      pltpu.sync_copy(stage_v, cacc.at[pl.ds(sid * ZROWS, ZROWS)])

      def obody(r, carry):
        ones_v[r, :] = jnp.ones((16,), jnp.float32)
        return carry

      lax.fori_loop(0, CH, obody, 0)
    pltpu.sync_copy(idx_hbm.at[wid], idx_v)
    plsc.subcore_barrier()

    def group(g, carry):
      j0 = g * GRP

      @pl.when(g > 0)
      def _drain_adds():
        for b in range(GRP):
          pltpu.make_async_copy(
              rows_v.at[b], acc.at[idx_v.at[j0 + b]], asem).wait()
          if with_cnt:
            pltpu.make_async_copy(
                ones_v, cacc.at[idx_v.at[j0 + b]], csem).wait()

      for b in range(GRP):
        pltpu.async_copy(
            msg_hbm.at[pl.ds(base + (j0 + b) * CH, CH)], rows_v.at[b], lsem)
      for b in range(GRP):
        pltpu.make_async_copy(
            msg_hbm.at[pl.ds(base, CH)], rows_v.at[b], lsem).wait()
      for b in range(GRP):
        pltpu.async_copy(rows_v.at[b], acc.at[idx_v.at[j0 + b]], asem,
                         add=True)
        if with_cnt:
          pltpu.async_copy(ones_v, cacc.at[idx_v.at[j0 + b]], csem, add=True)
      return carry

    lax.fori_loop(0, NGRP, group, 0)
    for b in range(GRP):
      pltpu.make_async_copy(rows_v.at[b], acc.at[idx_v.at[b]], asem).wait()
      if with_cnt:
        pltpu.make_async_copy(ones_v, cacc.at[idx_v.at[b]], csem).wait()
    plsc.subcore_barrier()

    pltpu.sync_copy(acc.at[pl.ds(sid * ZROWS, ZROWS)], stage_v)
    pltpu.sync_copy(stage_v, out_sum.at[cid, pl.ds(sid * ZROWS, ZROWS)])
    if with_cnt:
      pltpu.sync_copy(cacc.at[pl.ds(sid * ZROWS, ZROWS)], stage_v)
      pltpu.sync_copy(stage_v, out_cnt.at[cid, pl.ds(sid * ZROWS, ZROWS)])

  res = k(msg, idx3)
  return res if with_cnt else res[0]


_TB = 6400


def _tc_edge_msgs(eap, xjp, Wa_bd, ba8, Wb_bd, bb8, R_bd, S_bd):
  TB8 = _TB // 8

  def body(eap_ref, xj_ref, wa, ba_r, wb, bb_r, r_r, s_r, out_ref):
    hp = jnp.maximum(
        jnp.dot(eap_ref[...], wa[...], preferred_element_type=jnp.float32)
        + ba_r[...], 0.0)
    wp = jnp.dot(hp.astype(jnp.bfloat16), wb[...],
                 preferred_element_type=jnp.float32) + bb_r[...]
    xep = jnp.dot(xj_ref[...].astype(jnp.bfloat16), r_r[...],
                  preferred_element_type=jnp.float32)
    out_ref[...] = jnp.dot(
        (wp * xep).astype(jnp.bfloat16), s_r[...],
        preferred_element_type=jnp.float32)

  zero = lambda i: (0, 0)
  return pl.pallas_call(
      body,
      grid=(E // _TB,),
      in_specs=[
          pl.BlockSpec((TB8, 8 * EA), lambda i: (i, 0)),
          pl.BlockSpec((TB8, 128), lambda i: (i, 0)),
          pl.BlockSpec((8 * EA, 8 * 32), zero),
          pl.BlockSpec((1, 8 * 32), zero),
          pl.BlockSpec((8 * 32, 8 * IN * H), zero),
          pl.BlockSpec((1, 8 * IN * H), zero),
          pl.BlockSpec((128, 8 * IN * H), zero),
          pl.BlockSpec((8 * IN * H, 128), zero),
      ],
      out_specs=pl.BlockSpec((TB8, 128), lambda i: (i, 0)),
      out_shape=jax.ShapeDtypeStruct((E // 8, 128), jnp.float32),
  )(eap, xjp, Wa_bd, ba8.reshape(1, 8 * 32),
    Wb_bd.astype(jnp.bfloat16), bb8.reshape(1, 8 * IN * H),
    R_bd.astype(jnp.bfloat16), S_bd.astype(jnp.bfloat16))


NP8 = N // 8


def _tc_finalize1(sums_p, cnts_p, x_p, root_bd, bias_t):
  def body(s_ref, c_ref, x_ref, r_ref, b_ref, h_ref, rinv_ref):
    cnt = c_ref[0] + c_ref[1]
    rinv = 1.0 / jnp.maximum(cnt, 1.0)
    mean = (s_ref[0] + s_ref[1]) * rinv
    h = mean + jnp.dot(
        x_ref[...], r_ref[...], preferred_element_type=jnp.float32) + b_ref[...]
    h_ref[...] = jnp.maximum(h, 0.0)
    rinv_ref[...] = rinv

  return pl.pallas_call(
      body,
      out_shape=[
          jax.ShapeDtypeStruct((NP8, 128), jnp.float32),
          jax.ShapeDtypeStruct((NP8, 128), jnp.float32),
      ],
  )(sums_p, cnts_p, x_p, root_bd, bias_t.reshape(1, 128))


def _tc_finalize2(sums_p, rinv_p, h1_p, root_bd, bias_t, Wl_bd, bl_t):
  def body(s_ref, rinv_ref, h1_ref, r_ref, b_ref, wl_ref, bl_ref, out_ref):
    mean = (s_ref[0] + s_ref[1]) * rinv_ref[...]
    h2 = mean + jnp.dot(
        h1_ref[...], r_ref[...], preferred_element_type=jnp.float32) + b_ref[...]
    h2 = jnp.maximum(h2, 0.0)
    out_ref[...] = jnp.dot(
        h2, wl_ref[...], preferred_element_type=jnp.float32) + bl_ref[...]

  return pl.pallas_call(
      body,
      out_shape=jax.ShapeDtypeStruct((NP8, 8), jnp.float32),
  )(sums_p, rinv_p, h1_p, root_bd, bias_t.reshape(1, 128), Wl_bd,
    bl_t.reshape(1, 8))


def kernel(x, edge_index, edge_attr, W1a, b1a, W1b, b1b, root1, bias1,
           W2a, b2a, W2b, b2b, root2, bias2, Wl, bl):
  src3 = edge_index[0].astype(jnp.int32).reshape(NW, NCHUNK, CH)
  dst3 = edge_index[1].astype(jnp.int32).reshape(NW, NCHUNK, CH)

  c = jnp.arange(IN * H, dtype=jnp.int32)
  R = (jnp.arange(IN, dtype=jnp.int32)[:, None] == (c // H)[None, :]
       ).astype(jnp.float32)
  S = ((c % H)[:, None] == jnp.arange(H, dtype=jnp.int32)[None, :]
       ).astype(jnp.float32)

  eye8 = jnp.eye(8, dtype=jnp.float32)
  W1a_bd = jnp.kron(eye8, W1a)
  W1b_bd = jnp.kron(eye8, W1b)
  W2a_bd = jnp.kron(eye8, W2a)
  W2b_bd = jnp.kron(eye8, W2b)
  R_bd = jnp.kron(eye8, R)
  S_bd = jnp.kron(eye8, S)
  b1a8 = jnp.tile(b1a, 8)
  b1b8 = jnp.tile(b1b, 8)
  b2a8 = jnp.tile(b2a, 8)
  b2b8 = jnp.tile(b2b, 8)
  root1_bd = jnp.kron(eye8, root1)
  root2_bd = jnp.kron(eye8, root2)
  Wl_bd = jnp.kron(eye8, Wl)
  bias1_t = jnp.tile(bias1, 8)
  bias2_t = jnp.tile(bias2, 8)
  bl_t = jnp.tile(bl, 8)
  eap = edge_attr.reshape(E // 8, 8 * EA)

  xj = _sc_gather(x, src3)
  msg1 = _tc_edge_msgs(eap, xj.reshape(E // 8, 128),
                       W1a_bd, b1a8, W1b_bd, b1b8, R_bd, S_bd)
  sums1, cnts = _sc_scatter(msg1.reshape(E, 16), dst3, True)
  h1p, rinvp = _tc_finalize1(sums1.reshape(NC, NP8, 128),
                             cnts.reshape(NC, NP8, 128),
                             x.reshape(NP8, 128), root1_bd, bias1_t)

  xj2 = _sc_gather(h1p.reshape(N, H), src3)
  msg2 = _tc_edge_msgs(eap, xj2.reshape(E // 8, 128),
                       W2a_bd, b2a8, W2b_bd, b2b8, R_bd, S_bd)
  sums2 = _sc_scatter(msg2.reshape(E, 16), dst3, False)
  out = _tc_finalize2(sums2.reshape(NC, NP8, 128), rinvp, h1p,
                      root2_bd, bias2_t, Wl_bd, bl_t)
  return out.reshape(N)
